# Initial kernel scaffold; baseline (speedup 1.0000x reference)
import functools

import jax
import jax.numpy as jnp
from jax.experimental import pallas as pl
from jax.experimental.pallas import tpu as pltpu

_C = 16
_T = 200
_TPAD = 256


def _pick_block(n: int) -> int:
    for b in (2048, 2000, 1600, 1280, 1024, 1000, 800, 640, 512, 500, 400,
              256, 200, 128, 100, 64, 32, 16, 8):
        if n % b == 0:
            return b
    return n


def _metrics_body(nb, lab_ref, x_ref,
                  cm_out, map_out, prec_out, rec_out, recall_out,
                  precision_out, acc_out, loss_out,
                  cm_acc, pp_acc, tp_acc, loss_acc):
    i = pl.program_id(0)

    @pl.when(i == 0)
    def _init():
        cm_acc[...] = jnp.zeros_like(cm_acc)
        pp_acc[...] = jnp.zeros_like(pp_acc)
        tp_acc[...] = jnp.zeros_like(tp_acc)
        loss_acc[0, 0] = 0.0

    x = x_ref[...]
    lab = lab_ref[...]
    b = x.shape[0]

    iota_c_row = jax.lax.broadcasted_iota(jnp.int32, (1, _C), 1)
    ohl = (lab == iota_c_row).astype(jnp.float32)

    m = jnp.max(x, axis=1, keepdims=True)
    e = jnp.exp(x - m)
    z = jnp.sum(e, axis=1, keepdims=True)
    p = e / z
    s = p * jnp.float32(_T - 1)

    pm = jnp.max(p, axis=1, keepdims=True)
    ii = jax.lax.broadcasted_iota(jnp.int32, (b, _C), 1)
    pick = jnp.min(jnp.where(p == pm, ii, _C), axis=1, keepdims=True)
    ohp = (ii == pick).astype(jnp.float32)

    dn = (((0,), (0,)), ((), ()))
    cm_acc[...] += jax.lax.dot_general(ohl, ohp, dn,
                                       preferred_element_type=jnp.float32)

    thr = jax.lax.broadcasted_iota(jnp.float32, (1, _TPAD), 1)
    for c in range(_C):
        ge = (s[:, c:c + 1] >= thr).astype(jnp.float32)
        pp_acc[c:c + 1, :] += jnp.sum(ge, axis=0, keepdims=True)

    strue = jnp.sum(s * ohl, axis=1, keepdims=True)
    getrue = (strue >= thr).astype(jnp.float32)
    tp_acc[...] += jax.lax.dot_general(ohl, getrue, dn,
                                       preferred_element_type=jnp.float32)

    xt = jnp.sum(x * ohl, axis=1, keepdims=True)
    loss_acc[0, 0] += jnp.sum((m + jnp.log(z)) - xt)

    @pl.when(i == nb - 1)
    def _fin():
        cm = cm_acc[...]
        cm_out[...] = cm.astype(jnp.int32)
        eye = (jax.lax.broadcasted_iota(jnp.int32, (_C, _C), 0)
               == jax.lax.broadcasted_iota(jnp.int32, (_C, _C), 1))
        cmdiag = jnp.where(eye, cm, 0.0)
        tpd_col = jnp.sum(cmdiag, axis=1, keepdims=True)
        tpd_row = jnp.sum(cmdiag, axis=0, keepdims=True)
        support = jnp.sum(cm, axis=1, keepdims=True)
        predcnt = jnp.sum(cm, axis=0, keepdims=True)
        recall_per = tpd_col / jnp.maximum(support, 1.0)
        prec_per = tpd_row / jnp.maximum(predcnt, 1.0)
        recall = jnp.sum(recall_per) / _C
        precision = jnp.sum(prec_per) / _C
        recall_out[...] = jnp.reshape(recall, (1, 1))
        precision_out[...] = jnp.reshape(precision, (1, 1))
        acc_out[...] = jnp.reshape(recall, (1, 1))

        tp_curve = tp_acc[:, :_T]
        pp_curve = pp_acc[:, :_T]
        prec_curve = tp_curve / jnp.maximum(pp_curve, 1.0)
        rec_curve = tp_curve / jnp.maximum(support, 1.0)
        ones_col = jnp.ones((_C, 1), jnp.float32)
        zeros_col = jnp.zeros((_C, 1), jnp.float32)
        prec_out[...] = jnp.concatenate([prec_curve, ones_col], axis=1)
        rec_out[...] = jnp.concatenate([rec_curve, zeros_col], axis=1)
        rec_next = jnp.concatenate([rec_curve[:, 1:], zeros_col], axis=1)
        ap = jnp.sum((rec_curve - rec_next) * prec_curve, axis=1, keepdims=True)
        map_out[...] = jnp.reshape(jnp.sum(ap) / _C, (1, 1))
        loss_out[...] = jnp.reshape(loss_acc[0, 0] / (x.shape[0] * nb), (1, 1))


def kernel(logits, labels):
    n, c = logits.shape
    assert c == _C
    b = _pick_block(n)
    nb = n // b
    lab2 = labels.astype(jnp.int32).reshape(n, 1)

    f32 = jnp.float32
    out_shapes = (
        jax.ShapeDtypeStruct((_C, _C), jnp.int32),
        jax.ShapeDtypeStruct((1, 1), f32),
        jax.ShapeDtypeStruct((_C, _T + 1), f32),
        jax.ShapeDtypeStruct((_C, _T + 1), f32),
        jax.ShapeDtypeStruct((1, 1), f32),
        jax.ShapeDtypeStruct((1, 1), f32),
        jax.ShapeDtypeStruct((1, 1), f32),
        jax.ShapeDtypeStruct((1, 1), f32),
    )

    def whole(r, c_):
        return pl.BlockSpec((r, c_), lambda i: (0, 0))

    grid_spec = pltpu.PrefetchScalarGridSpec(
        num_scalar_prefetch=0,
        grid=(nb,),
        in_specs=[
            pl.BlockSpec((b, 1), lambda i: (i, 0)),
            pl.BlockSpec((b, _C), lambda i: (i, 0)),
        ],
        out_specs=[
            whole(_C, _C), whole(1, 1), whole(_C, _T + 1),
            whole(_C, _T + 1), whole(1, 1), whole(1, 1),
            whole(1, 1), whole(1, 1),
        ],
        scratch_shapes=[
            pltpu.VMEM((_C, _C), f32),
            pltpu.VMEM((_C, _TPAD), f32),
            pltpu.VMEM((_C, _TPAD), f32),
            pltpu.SMEM((1, 1), f32),
        ],
    )
    outs = pl.pallas_call(
        functools.partial(_metrics_body, nb),
        grid_spec=grid_spec,
        out_shape=out_shapes,
    )(lab2, logits)
    cm, map_, prec_full, rec_full, recall, precision, accuracy, loss = outs
    thresholds = jnp.linspace(0.0, 1.0, _T)
    return (cm, map_.reshape(()), prec_full, rec_full, thresholds,
            recall.reshape(()), precision.reshape(()), accuracy.reshape(()),
            loss.reshape(()))


# speedup vs baseline: 8.6788x; 8.6788x over previous
import functools

import jax
import jax.numpy as jnp
from jax.experimental import pallas as pl
from jax.experimental.pallas import tpu as pltpu

_C = 16
_T = 200
_TPAD = 256


def _pick_block(n: int) -> int:
    for b in (2048, 2000, 1600, 1280, 1024, 1000, 800, 640, 512, 500, 400,
              256, 200, 128, 100, 64, 32, 16, 8):
        if n % b == 0:
            return b
    return n


def _metrics_body(nb, lab_ref, x_ref,
                  cm_out, map_out, prec_out, rec_out, recall_out,
                  precision_out, acc_out, loss_out,
                  cm_acc, pp_acc, tp_acc, loss_acc):
    i = pl.program_id(0)

    @pl.when(i == 0)
    def _init():
        cm_acc[...] = jnp.zeros_like(cm_acc)
        pp_acc[...] = jnp.zeros_like(pp_acc)
        tp_acc[...] = jnp.zeros_like(tp_acc)
        loss_acc[0, 0] = 0.0

    x = x_ref[...]
    lab = lab_ref[...]
    b = x.shape[0]

    iota_c_row = jax.lax.broadcasted_iota(jnp.int32, (1, _C), 1)
    ohl = (lab == iota_c_row).astype(jnp.float32)

    m = jnp.max(x, axis=1, keepdims=True)
    e = jnp.exp(x - m)
    z = jnp.sum(e, axis=1, keepdims=True)
    p = e / z
    s = p * jnp.float32(_T - 1)

    pm = jnp.max(p, axis=1, keepdims=True)
    ii = jax.lax.broadcasted_iota(jnp.int32, (b, _C), 1)
    pick = jnp.min(jnp.where(p == pm, ii, _C), axis=1, keepdims=True)
    ohp = (ii == pick).astype(jnp.float32)

    dn = (((0,), (0,)), ((), ()))
    cm_acc[...] += jax.lax.dot_general(ohl, ohp, dn,
                                       preferred_element_type=jnp.float32)

    thr = jax.lax.broadcasted_iota(jnp.int32, (1, _TPAD), 1).astype(jnp.float32)
    for c in range(_C):
        ge = (s[:, c:c + 1] >= thr).astype(jnp.float32)
        pp_acc[c:c + 1, :] += jnp.sum(ge, axis=0, keepdims=True)

    strue = jnp.sum(s * ohl, axis=1, keepdims=True)
    getrue = (strue >= thr).astype(jnp.float32)
    tp_acc[...] += jax.lax.dot_general(ohl, getrue, dn,
                                       preferred_element_type=jnp.float32)

    xt = jnp.sum(x * ohl, axis=1, keepdims=True)
    loss_acc[0, 0] += jnp.sum((m + jnp.log(z)) - xt)

    @pl.when(i == nb - 1)
    def _fin():
        cm = cm_acc[...]
        cm_out[...] = cm.astype(jnp.int32)
        eye = (jax.lax.broadcasted_iota(jnp.int32, (_C, _C), 0)
               == jax.lax.broadcasted_iota(jnp.int32, (_C, _C), 1))
        cmdiag = jnp.where(eye, cm, 0.0)
        tpd_col = jnp.sum(cmdiag, axis=1, keepdims=True)
        tpd_row = jnp.sum(cmdiag, axis=0, keepdims=True)
        support = jnp.sum(cm, axis=1, keepdims=True)
        predcnt = jnp.sum(cm, axis=0, keepdims=True)
        recall_per = tpd_col / jnp.maximum(support, 1.0)
        prec_per = tpd_row / jnp.maximum(predcnt, 1.0)
        recall = jnp.sum(recall_per) / _C
        precision = jnp.sum(prec_per) / _C
        recall_out[...] = jnp.reshape(recall, (1, 1))
        precision_out[...] = jnp.reshape(precision, (1, 1))
        acc_out[...] = jnp.reshape(recall, (1, 1))

        tp_curve = tp_acc[:, :_T]
        pp_curve = pp_acc[:, :_T]
        prec_curve = tp_curve / jnp.maximum(pp_curve, 1.0)
        rec_curve = tp_curve / jnp.maximum(support, 1.0)
        ones_col = jnp.ones((_C, 1), jnp.float32)
        zeros_col = jnp.zeros((_C, 1), jnp.float32)
        prec_out[...] = jnp.concatenate([prec_curve, ones_col], axis=1)
        rec_out[...] = jnp.concatenate([rec_curve, zeros_col], axis=1)
        rec_next = jnp.concatenate([rec_curve[:, 1:], zeros_col], axis=1)
        ap = jnp.sum((rec_curve - rec_next) * prec_curve, axis=1, keepdims=True)
        map_out[...] = jnp.reshape(jnp.sum(ap) / _C, (1, 1))
        loss_out[...] = jnp.reshape(loss_acc[0, 0] / (x.shape[0] * nb), (1, 1))


def kernel(logits, labels):
    n, c = logits.shape
    assert c == _C
    b = _pick_block(n)
    nb = n // b
    lab2 = labels.astype(jnp.int32).reshape(n, 1)

    f32 = jnp.float32
    out_shapes = (
        jax.ShapeDtypeStruct((_C, _C), jnp.int32),
        jax.ShapeDtypeStruct((1, 1), f32),
        jax.ShapeDtypeStruct((_C, _T + 1), f32),
        jax.ShapeDtypeStruct((_C, _T + 1), f32),
        jax.ShapeDtypeStruct((1, 1), f32),
        jax.ShapeDtypeStruct((1, 1), f32),
        jax.ShapeDtypeStruct((1, 1), f32),
        jax.ShapeDtypeStruct((1, 1), f32),
    )

    def whole(r, c_):
        return pl.BlockSpec((r, c_), lambda i: (0, 0))

    grid_spec = pltpu.PrefetchScalarGridSpec(
        num_scalar_prefetch=0,
        grid=(nb,),
        in_specs=[
            pl.BlockSpec((b, 1), lambda i: (i, 0)),
            pl.BlockSpec((b, _C), lambda i: (i, 0)),
        ],
        out_specs=[
            whole(_C, _C), whole(1, 1), whole(_C, _T + 1),
            whole(_C, _T + 1), whole(1, 1), whole(1, 1),
            whole(1, 1), whole(1, 1),
        ],
        scratch_shapes=[
            pltpu.VMEM((_C, _C), f32),
            pltpu.VMEM((_C, _TPAD), f32),
            pltpu.VMEM((_C, _TPAD), f32),
            pltpu.SMEM((1, 1), f32),
        ],
    )
    outs = pl.pallas_call(
        functools.partial(_metrics_body, nb),
        grid_spec=grid_spec,
        out_shape=out_shapes,
    )(lab2, logits)
    cm, map_, prec_full, rec_full, recall, precision, accuracy, loss = outs
    thresholds = jnp.linspace(0.0, 1.0, _T)
    return (cm, map_.reshape(()), prec_full, rec_full, thresholds,
            recall.reshape(()), precision.reshape(()), accuracy.reshape(()),
            loss.reshape(()))


# speedup vs baseline: 14.7212x; 1.6962x over previous
import functools

import jax
import jax.numpy as jnp
from jax import lax
from jax.experimental import pallas as pl
from jax.experimental.pallas import tpu as pltpu
from jax.experimental.pallas import tpu_sc as plsc

_C = 16
_T = 200
_TPAD = 256
_HP = _C * (_T + 1)


def _pick_block(n: int) -> int:
    for b in (2048, 2000, 1600, 1280, 1024, 1000, 800, 640, 512, 500, 400,
              256, 200, 128, 100, 64, 32, 16, 8):
        if n % b == 0:
            return b
    return n


def _stage1_body(nb, lab_ref, x_ref,
                 bins_out, cm_out, tp_out, loss_out,
                 cm_acc, tp_acc, loss_acc):
    i = pl.program_id(0)

    @pl.when(i == 0)
    def _init():
        cm_acc[...] = jnp.zeros_like(cm_acc)
        tp_acc[...] = jnp.zeros_like(tp_acc)
        loss_acc[0, 0] = 0.0

    x = x_ref[...]
    lab = lab_ref[...]
    b = x.shape[0]

    iota_c_row = jax.lax.broadcasted_iota(jnp.int32, (1, _C), 1)
    ohl = (lab == iota_c_row).astype(jnp.float32)

    m = jnp.max(x, axis=1, keepdims=True)
    e = jnp.exp(x - m)
    z = jnp.sum(e, axis=1, keepdims=True)
    p = e / z
    s = p * jnp.float32(_T - 1)

    bins_out[...] = jnp.clip(s.astype(jnp.int32), 0, _T - 1)

    pm = jnp.max(p, axis=1, keepdims=True)
    ii = jax.lax.broadcasted_iota(jnp.int32, (b, _C), 1)
    pick = jnp.min(jnp.where(p == pm, ii, _C), axis=1, keepdims=True)
    ohp = (ii == pick).astype(jnp.float32)

    dn = (((0,), (0,)), ((), ()))
    cm_acc[...] += jax.lax.dot_general(ohl, ohp, dn,
                                       preferred_element_type=jnp.float32)

    thr = jax.lax.broadcasted_iota(jnp.int32, (1, _TPAD), 1).astype(jnp.float32)
    strue = jnp.sum(s * ohl, axis=1, keepdims=True)
    getrue = (strue >= thr).astype(jnp.float32)
    tp_acc[...] += jax.lax.dot_general(ohl, getrue, dn,
                                       preferred_element_type=jnp.float32)

    xt = jnp.sum(x * ohl, axis=1, keepdims=True)
    loss_acc[0, 0] += jnp.sum((m + jnp.log(z)) - xt)

    @pl.when(i == nb - 1)
    def _fin():
        cm_out[...] = cm_acc[...].astype(jnp.int32)
        tp_out[...] = tp_acc[...]
        loss_out[...] = jnp.reshape(loss_acc[0, 0], (1, 1))


def _stage1(logits, lab2):
    n, _ = logits.shape
    b = _pick_block(n)
    nb = n // b
    f32 = jnp.float32
    grid_spec = pltpu.PrefetchScalarGridSpec(
        num_scalar_prefetch=0,
        grid=(nb,),
        in_specs=[
            pl.BlockSpec((b, 1), lambda i: (i, 0)),
            pl.BlockSpec((b, _C), lambda i: (i, 0)),
        ],
        out_specs=[
            pl.BlockSpec((b, _C), lambda i: (i, 0)),
            pl.BlockSpec((_C, _C), lambda i: (0, 0)),
            pl.BlockSpec((_C, _TPAD), lambda i: (0, 0)),
            pl.BlockSpec((1, 1), lambda i: (0, 0)),
        ],
        scratch_shapes=[
            pltpu.VMEM((_C, _C), f32),
            pltpu.VMEM((_C, _TPAD), f32),
            pltpu.SMEM((1, 1), f32),
        ],
    )
    return pl.pallas_call(
        functools.partial(_stage1_body, nb),
        grid_spec=grid_spec,
        out_shape=(
            jax.ShapeDtypeStruct((n, _C), jnp.int32),
            jax.ShapeDtypeStruct((_C, _C), jnp.int32),
            jax.ShapeDtypeStruct((_C, _TPAD), f32),
            jax.ShapeDtypeStruct((1, 1), f32),
        ),
    )(lab2, logits)


def _sc_hist(bins, n):
    info = plsc.get_sparse_core_info()
    nc, ns = info.num_cores, info.num_subcores
    nw = nc * ns
    chunk = 1000
    assert n % chunk == 0 and chunk % 8 == 0
    nchunks_total = n // chunk
    base_count = nchunks_total // nw
    extra = nchunks_total - base_count * nw
    mesh = plsc.VectorSubcoreMesh(core_axis_name="c", subcore_axis_name="s")

    @functools.partial(
        pl.kernel, mesh=mesh,
        compiler_params=pltpu.CompilerParams(needs_layout_passes=False),
        out_type=jax.ShapeDtypeStruct((nw * _HP,), jnp.float32),
        scratch_types=[
            pltpu.VMEM((chunk * _C,), jnp.int32),
            pltpu.VMEM((_HP,), jnp.float32),
        ],
    )
    def sc_hist(bins_hbm, out_hbm, buf, hist):
        wid = lax.axis_index("s") * nc + lax.axis_index("c")

        zeros16 = jnp.zeros((16,), jnp.float32)

        def zb(t, carry):
            hist[pl.ds(t * 16, 16)] = zeros16
            return carry

        lax.fori_loop(0, _HP // 16, zb, 0)

        offs = lax.iota(jnp.int32, 16) * (_T + 1)
        ones = jnp.ones((16,), jnp.float32)

        def chunk_body(j, carry):
            k = wid + j * nw
            pltpu.sync_copy(bins_hbm.at[pl.ds(k * chunk * _C, chunk * _C)], buf)

            def body(i, c2):
                idx = buf[pl.ds(i * _C, _C)] + offs
                plsc.addupdate_scatter(hist, [idx], ones)
                return c2

            lax.fori_loop(0, chunk, body, 0, unroll=8)
            return carry

        cnt = jnp.where(wid < extra, base_count + 1, base_count)
        lax.fori_loop(0, cnt, chunk_body, 0)
        pltpu.sync_copy(hist, out_hbm.at[pl.ds(wid * _HP, _HP)])

    return sc_hist(bins)


def _stage3_body(n, hist_ref, cm_ref, tp_ref, losssum_ref,
                 map_out, prec_out, rec_out, recall_out,
                 precision_out, acc_out, loss_out):
    totals = jnp.sum(hist_ref[...], axis=0)
    incl = totals
    k = 1
    while k < _T + 1:
        pad = jnp.zeros((_C, k), jnp.float32)
        incl = incl + jnp.concatenate([pad, incl[:, :-k]], axis=1)
        k *= 2
    ppc = jnp.float32(n) - incl + totals
    pp_curve = ppc[:, :_T]

    cm = cm_ref[...].astype(jnp.float32)
    eye = (jax.lax.broadcasted_iota(jnp.int32, (_C, _C), 0)
           == jax.lax.broadcasted_iota(jnp.int32, (_C, _C), 1))
    cmdiag = jnp.where(eye, cm, 0.0)
    tpd_col = jnp.sum(cmdiag, axis=1, keepdims=True)
    tpd_row = jnp.sum(cmdiag, axis=0, keepdims=True)
    support = jnp.sum(cm, axis=1, keepdims=True)
    predcnt = jnp.sum(cm, axis=0, keepdims=True)
    recall_per = tpd_col / jnp.maximum(support, 1.0)
    prec_per = tpd_row / jnp.maximum(predcnt, 1.0)
    recall = jnp.sum(recall_per) / _C
    precision = jnp.sum(prec_per) / _C
    recall_out[...] = jnp.reshape(recall, (1, 1))
    precision_out[...] = jnp.reshape(precision, (1, 1))
    acc_out[...] = jnp.reshape(recall, (1, 1))

    tp_curve = tp_ref[:, :_T]
    prec_curve = tp_curve / jnp.maximum(pp_curve, 1.0)
    rec_curve = tp_curve / jnp.maximum(support, 1.0)
    ones_col = jnp.ones((_C, 1), jnp.float32)
    zeros_col = jnp.zeros((_C, 1), jnp.float32)
    prec_out[...] = jnp.concatenate([prec_curve, ones_col], axis=1)
    rec_out[...] = jnp.concatenate([rec_curve, zeros_col], axis=1)
    rec_next = jnp.concatenate([rec_curve[:, 1:], zeros_col], axis=1)
    ap = jnp.sum((rec_curve - rec_next) * prec_curve, axis=1, keepdims=True)
    map_out[...] = jnp.reshape(jnp.sum(ap) / _C, (1, 1))
    loss_out[...] = losssum_ref[...] / jnp.float32(n)


def _stage3(hist3, cm, tp, loss_sum, n, nw):
    f32 = jnp.float32
    one = lambda: pl.BlockSpec((1, 1), lambda: (0, 0))
    outs = pl.pallas_call(
        functools.partial(_stage3_body, n),
        grid=(),
        in_specs=[
            pl.BlockSpec((nw, _C, _T + 1), lambda: (0, 0, 0)),
            pl.BlockSpec((_C, _C), lambda: (0, 0)),
            pl.BlockSpec((_C, _TPAD), lambda: (0, 0)),
            one(),
        ],
        out_specs=[
            one(), pl.BlockSpec((_C, _T + 1), lambda: (0, 0)),
            pl.BlockSpec((_C, _T + 1), lambda: (0, 0)),
            one(), one(), one(), one(),
        ],
        out_shape=(
            jax.ShapeDtypeStruct((1, 1), f32),
            jax.ShapeDtypeStruct((_C, _T + 1), f32),
            jax.ShapeDtypeStruct((_C, _T + 1), f32),
            jax.ShapeDtypeStruct((1, 1), f32),
            jax.ShapeDtypeStruct((1, 1), f32),
            jax.ShapeDtypeStruct((1, 1), f32),
            jax.ShapeDtypeStruct((1, 1), f32),
        ),
    )(hist3, cm, tp, loss_sum)
    return outs


def kernel(logits, labels):
    n, c = logits.shape
    assert c == _C
    lab2 = labels.astype(jnp.int32).reshape(n, 1)

    bins, cm, tp, loss_sum = _stage1(logits, lab2)
    parts = _sc_hist(bins.reshape(-1), n)
    nw = parts.shape[0] // _HP
    hist3 = parts.reshape(nw, _C, _T + 1)
    (map_, prec_full, rec_full, recall, precision, accuracy,
     loss) = _stage3(hist3, cm, tp, loss_sum, n, nw)

    thresholds = jnp.linspace(0.0, 1.0, _T)
    return (cm, map_.reshape(()), prec_full, rec_full, thresholds,
            recall.reshape(()), precision.reshape(()), accuracy.reshape(()),
            loss.reshape(()))


# speedup vs baseline: 31.8902x; 2.1663x over previous
import functools

import jax
import jax.numpy as jnp
from jax import lax
from jax.experimental import pallas as pl
from jax.experimental.pallas import tpu as pltpu
from jax.experimental.pallas import tpu_sc as plsc

_C = 16
_T = 200
_HP = _C * (_T + 1)
_LS = 2 * _HP


def _pick_block(n: int) -> int:
    for b in (4000, 2500, 2000, 1600, 1000, 800, 500, 400, 200, 100, 50, 25, 8):
        if n % b == 0:
            return b
    return n


def _stage1_body(nb, lab_ref, x_ref,
                 bins_out, tpidx_out, cm_out, loss_out,
                 cm_acc, loss_acc):
    i = pl.program_id(0)

    @pl.when(i == 0)
    def _init():
        cm_acc[...] = jnp.zeros_like(cm_acc)
        loss_acc[0, 0] = 0.0

    x = x_ref[0]
    lab = lab_ref[0]
    b = x.shape[1]

    m = jnp.max(x, axis=0, keepdims=True)
    e = jnp.exp(x - m)
    z = jnp.sum(e, axis=0, keepdims=True)
    inv = jnp.float32(_T - 1) / z
    s = e * inv
    bins = jnp.clip(s.astype(jnp.int32), 0, _T - 1)
    bins_out[...] = bins[None]

    em = jnp.max(e, axis=0, keepdims=True)
    iota_c = jax.lax.broadcasted_iota(jnp.int32, (_C, b), 0)
    pick = jnp.min(jnp.where(e == em, iota_c, _C), axis=0, keepdims=True)
    ohp = (iota_c == pick).astype(jnp.float32)
    ohl = lab == iota_c
    dn = (((1,), (1,)), ((), ()))
    cm_acc[...] += jax.lax.dot_general(ohl.astype(jnp.float32), ohp, dn,
                                       preferred_element_type=jnp.float32)

    bins_true = jnp.sum(jnp.where(ohl, bins, 0), axis=0, keepdims=True)
    tpidx_out[...] = (lab * (_T + 1) + bins_true)[None]

    xt = jnp.sum(jnp.where(ohl, x, 0.0), axis=0, keepdims=True)
    loss_acc[0, 0] += jnp.sum((m + jnp.log(z)) - xt)

    @pl.when(i == nb - 1)
    def _fin():
        cm_out[...] = cm_acc[...].astype(jnp.int32)
        loss_out[...] = jnp.reshape(loss_acc[0, 0], (1, 1))


def _stage1(x3, lab3):
    nb, _, b = x3.shape
    f32 = jnp.float32
    grid_spec = pltpu.PrefetchScalarGridSpec(
        num_scalar_prefetch=0,
        grid=(nb,),
        in_specs=[
            pl.BlockSpec((1, 1, b), lambda i: (i, 0, 0)),
            pl.BlockSpec((1, _C, b), lambda i: (i, 0, 0)),
        ],
        out_specs=[
            pl.BlockSpec((1, _C, b), lambda i: (i, 0, 0)),
            pl.BlockSpec((1, 1, b), lambda i: (i, 0, 0)),
            pl.BlockSpec((_C, _C), lambda i: (0, 0)),
            pl.BlockSpec((1, 1), lambda i: (0, 0)),
        ],
        scratch_shapes=[
            pltpu.VMEM((_C, _C), f32),
            pltpu.SMEM((1, 1), f32),
        ],
    )
    return pl.pallas_call(
        functools.partial(_stage1_body, nb),
        grid_spec=grid_spec,
        out_shape=(
            jax.ShapeDtypeStruct((nb, _C, b), jnp.int32),
            jax.ShapeDtypeStruct((nb, 1, b), jnp.int32),
            jax.ShapeDtypeStruct((_C, _C), jnp.int32),
            jax.ShapeDtypeStruct((1, 1), f32),
        ),
    )(lab3, x3)


def _sc_hist(bins_flat, tpidx, n, blk):
    info = plsc.get_sparse_core_info()
    nc, ns = info.num_cores, info.num_subcores
    nw = nc * ns
    w = 800
    assert n % w == 0 and w % 16 == 0 and blk % w == 0
    rpb = blk // w
    nchunks_total = n // w
    base_count = nchunks_total // nw
    extra = nchunks_total - base_count * nw
    mesh = plsc.VectorSubcoreMesh(core_axis_name="c", subcore_axis_name="s")

    @functools.partial(
        pl.kernel, mesh=mesh,
        compiler_params=pltpu.CompilerParams(needs_layout_passes=False),
        out_type=jax.ShapeDtypeStruct((nw * 16 * _LS,), jnp.float32),
        scratch_types=[
            pltpu.VMEM((_C * w,), jnp.int32),
            pltpu.VMEM((w,), jnp.int32),
            pltpu.VMEM((16 * _LS,), jnp.float32),
        ],
    )
    def sc_hist(bins_hbm, tpidx_hbm, out_hbm, bufpp, buftp, hist):
        wid = lax.axis_index("s") * nc + lax.axis_index("c")

        zeros16 = jnp.zeros((16,), jnp.float32)

        def zb(t, carry):
            hist[pl.ds(t * 16, 16)] = zeros16
            return carry

        lax.fori_loop(0, 16 * _LS // 16, zb, 0)

        lane_offs = lax.iota(jnp.int32, 16) * _LS
        ones = jnp.ones((16,), jnp.float32)

        def chunk_body(j, carry):
            k = wid + j * nw
            blk_i = k // rpb
            src0 = (blk_i * _C) * blk + (k % rpb) * w
            for c in range(_C):
                pltpu.sync_copy(bins_hbm.at[pl.ds(src0 + c * blk, w)],
                                bufpp.at[pl.ds(c * w, w)])
            pltpu.sync_copy(tpidx_hbm.at[pl.ds(k * w, w)], buftp)

            for c in range(_C):
                offs_c = lane_offs + (c * (_T + 1))

                def body_pp(v, c2, offs_c=offs_c, c=c):
                    idx = bufpp[pl.ds(c * w + v * 16, 16)] + offs_c
                    plsc.addupdate_scatter(hist, [idx], ones)
                    return c2

                lax.fori_loop(0, w // 16, body_pp, 0, unroll=10)

            offs_tp = lane_offs + _HP

            def body_tp(v, c2):
                idx = buftp[pl.ds(v * 16, 16)] + offs_tp
                plsc.addupdate_scatter(hist, [idx], ones)
                return c2

            lax.fori_loop(0, w // 16, body_tp, 0, unroll=10)
            return carry

        cnt = jnp.where(wid < extra, base_count + 1, base_count)
        lax.fori_loop(0, cnt, chunk_body, 0)
        pltpu.sync_copy(hist, out_hbm.at[pl.ds(wid * 16 * _LS, 16 * _LS)])

    return sc_hist(bins_flat, tpidx)


def _excl_rev_cumsum_ge(x, total):
    incl = x
    k = 1
    while k < _T + 1:
        pad = jnp.zeros((_C, k), jnp.float32)
        incl = incl + jnp.concatenate([pad, incl[:, :-k]], axis=1)
        k *= 2
    return total - incl + x


def _stage3_body(n, pp_ref, tp_ref, cm_ref, losssum_ref,
                 map_out, prec_out, rec_out, recall_out,
                 precision_out, acc_out, loss_out):
    cm = cm_ref[...].astype(jnp.float32)
    eye = (jax.lax.broadcasted_iota(jnp.int32, (_C, _C), 0)
           == jax.lax.broadcasted_iota(jnp.int32, (_C, _C), 1))
    cmdiag = jnp.where(eye, cm, 0.0)
    tpd_col = jnp.sum(cmdiag, axis=1, keepdims=True)
    tpd_row = jnp.sum(cmdiag, axis=0, keepdims=True)
    support = jnp.sum(cm, axis=1, keepdims=True)
    predcnt = jnp.sum(cm, axis=0, keepdims=True)
    recall_per = tpd_col / jnp.maximum(support, 1.0)
    prec_per = tpd_row / jnp.maximum(predcnt, 1.0)
    recall = jnp.sum(recall_per) / _C
    precision = jnp.sum(prec_per) / _C
    recall_out[...] = jnp.reshape(recall, (1, 1))
    precision_out[...] = jnp.reshape(precision, (1, 1))
    acc_out[...] = jnp.reshape(recall, (1, 1))

    totals = jnp.sum(pp_ref[...], axis=0)
    tp_totals = jnp.sum(tp_ref[...], axis=0)
    pos_total = jnp.sum(tp_totals, axis=1, keepdims=True)
    pp_curve = _excl_rev_cumsum_ge(totals, jnp.float32(n))[:, :_T]
    tp_curve = _excl_rev_cumsum_ge(tp_totals, pos_total)[:, :_T]

    prec_curve = tp_curve / jnp.maximum(pp_curve, 1.0)
    rec_curve = tp_curve / jnp.maximum(pos_total, 1.0)
    ones_col = jnp.ones((_C, 1), jnp.float32)
    zeros_col = jnp.zeros((_C, 1), jnp.float32)
    prec_out[...] = jnp.concatenate([prec_curve, ones_col], axis=1)
    rec_out[...] = jnp.concatenate([rec_curve, zeros_col], axis=1)
    rec_next = jnp.concatenate([rec_curve[:, 1:], zeros_col], axis=1)
    ap = jnp.sum((rec_curve - rec_next) * prec_curve, axis=1, keepdims=True)
    map_out[...] = jnp.reshape(jnp.sum(ap) / _C, (1, 1))
    loss_out[...] = losssum_ref[...] / jnp.float32(n)


def _stage3(pp4, tp4, cm, loss_sum, n):
    f32 = jnp.float32
    npart = pp4.shape[0]
    one = lambda: pl.BlockSpec((1, 1), lambda: (0, 0))
    return pl.pallas_call(
        functools.partial(_stage3_body, n),
        grid=(),
        in_specs=[
            pl.BlockSpec((npart, _C, _T + 1), lambda: (0, 0, 0)),
            pl.BlockSpec((npart, _C, _T + 1), lambda: (0, 0, 0)),
            pl.BlockSpec((_C, _C), lambda: (0, 0)),
            one(),
        ],
        out_specs=[
            one(), pl.BlockSpec((_C, _T + 1), lambda: (0, 0)),
            pl.BlockSpec((_C, _T + 1), lambda: (0, 0)),
            one(), one(), one(), one(),
        ],
        out_shape=(
            jax.ShapeDtypeStruct((1, 1), f32),
            jax.ShapeDtypeStruct((_C, _T + 1), f32),
            jax.ShapeDtypeStruct((_C, _T + 1), f32),
            jax.ShapeDtypeStruct((1, 1), f32),
            jax.ShapeDtypeStruct((1, 1), f32),
            jax.ShapeDtypeStruct((1, 1), f32),
            jax.ShapeDtypeStruct((1, 1), f32),
        ),
    )(pp4, tp4, cm, loss_sum)


def kernel(logits, labels):
    n, c = logits.shape
    assert c == _C
    b = _pick_block(n)
    nb = n // b
    x3 = logits.reshape(nb, b, _C).transpose(0, 2, 1)
    lab3 = labels.astype(jnp.int32).reshape(nb, 1, b)

    bins3, tpidx3, cm, loss_sum = _stage1(x3, lab3)
    parts = _sc_hist(bins3.reshape(-1), tpidx3.reshape(-1), n, b)
    nw = parts.shape[0] // (16 * _LS)
    parts5 = parts.reshape(nw * 16, 2, _C, _T + 1)
    pp4 = parts5[:, 0]
    tp4 = parts5[:, 1]
    (map_, prec_full, rec_full, recall, precision, accuracy,
     loss) = _stage3(pp4, tp4, cm, loss_sum, n)

    thresholds = jnp.linspace(0.0, 1.0, _T)
    return (cm, map_.reshape(()), prec_full, rec_full, thresholds,
            recall.reshape(()), precision.reshape(()), accuracy.reshape(()),
            loss.reshape(()))


# speedup vs baseline: 47.9166x; 1.5026x over previous
import functools

import jax
import jax.numpy as jnp
from jax import lax
from jax.experimental import pallas as pl
from jax.experimental.pallas import tpu as pltpu
from jax.experimental.pallas import tpu_sc as plsc

_C = 16
_T = 200
_HP = _C * (_T + 1)
_LS = 2 * _HP


def _pick_block(n: int) -> int:
    for b in (4000, 2500, 2000, 1600, 1000, 800, 500, 400, 200, 100, 50, 25, 8):
        if n % b == 0:
            return b
    return n


def _stage1_body(nb, lab_ref, x_ref,
                 bins_out, tpidx_out, cm_out, loss_out,
                 cm_acc, loss_acc):
    i = pl.program_id(0)

    @pl.when(i == 0)
    def _init():
        cm_acc[...] = jnp.zeros_like(cm_acc)
        loss_acc[0, 0] = 0.0

    x = x_ref[0]
    lab = lab_ref[0]
    b = x.shape[1]

    m = jnp.max(x, axis=0, keepdims=True)
    e = jnp.exp(x - m)
    z = jnp.sum(e, axis=0, keepdims=True)
    inv = jnp.float32(_T - 1) / z
    s = e * inv
    bins = jnp.clip(s.astype(jnp.int32), 0, _T - 1)
    bins_out[...] = bins[None]

    em = jnp.max(e, axis=0, keepdims=True)
    iota_c = jax.lax.broadcasted_iota(jnp.int32, (_C, b), 0)
    pick = jnp.min(jnp.where(e == em, iota_c, _C), axis=0, keepdims=True)
    ohp = (iota_c == pick).astype(jnp.float32)
    ohl = lab == iota_c
    dn = (((1,), (1,)), ((), ()))
    cm_acc[...] += jax.lax.dot_general(ohl.astype(jnp.float32), ohp, dn,
                                       preferred_element_type=jnp.float32)

    bins_true = jnp.sum(jnp.where(ohl, bins, 0), axis=0, keepdims=True)
    tpidx_out[...] = (lab * (_T + 1) + bins_true)[None]

    xt = jnp.sum(jnp.where(ohl, x, 0.0), axis=0, keepdims=True)
    loss_acc[0, 0] += jnp.sum((m + jnp.log(z)) - xt)

    @pl.when(i == nb - 1)
    def _fin():
        cm_out[...] = cm_acc[...].astype(jnp.int32)
        loss_out[...] = jnp.reshape(loss_acc[0, 0], (1, 1))


def _stage1(x3, lab3):
    nb, _, b = x3.shape
    f32 = jnp.float32
    grid_spec = pltpu.PrefetchScalarGridSpec(
        num_scalar_prefetch=0,
        grid=(nb,),
        in_specs=[
            pl.BlockSpec((1, 1, b), lambda i: (i, 0, 0)),
            pl.BlockSpec((1, _C, b), lambda i: (i, 0, 0)),
        ],
        out_specs=[
            pl.BlockSpec((1, _C, b), lambda i: (i, 0, 0)),
            pl.BlockSpec((1, 1, b), lambda i: (i, 0, 0)),
            pl.BlockSpec((_C, _C), lambda i: (0, 0)),
            pl.BlockSpec((1, 1), lambda i: (0, 0)),
        ],
        scratch_shapes=[
            pltpu.VMEM((_C, _C), f32),
            pltpu.SMEM((1, 1), f32),
        ],
    )
    return pl.pallas_call(
        functools.partial(_stage1_body, nb),
        grid_spec=grid_spec,
        out_shape=(
            jax.ShapeDtypeStruct((nb, _C, b), jnp.int32),
            jax.ShapeDtypeStruct((nb, 1, b), jnp.int32),
            jax.ShapeDtypeStruct((_C, _C), jnp.int32),
            jax.ShapeDtypeStruct((1, 1), f32),
        ),
    )(lab3, x3)


def _sc_hist(bins_flat, tpidx, n, blk):
    info = plsc.get_sparse_core_info()
    nc, ns = info.num_cores, info.num_subcores
    nw = nc * ns
    w = 800
    assert n % w == 0 and w % 16 == 0 and blk % w == 0
    rpb = blk // w
    nchunks_total = n // w
    base_count = nchunks_total // nw
    extra = nchunks_total - base_count * nw
    mesh = plsc.VectorSubcoreMesh(core_axis_name="c", subcore_axis_name="s")

    @functools.partial(
        pl.kernel, mesh=mesh,
        compiler_params=pltpu.CompilerParams(needs_layout_passes=False),
        out_type=jax.ShapeDtypeStruct((nw * 16 * _LS,), jnp.float32),
        scratch_types=[
            pltpu.VMEM((_C * w,), jnp.int32),
            pltpu.VMEM((_C * w,), jnp.int32),
            pltpu.VMEM((w,), jnp.int32),
            pltpu.VMEM((w,), jnp.int32),
            pltpu.VMEM((16 * _LS,), jnp.float32),
            pltpu.SemaphoreType.DMA,
            pltpu.SemaphoreType.DMA,
        ],
    )
    def sc_hist(bins_hbm, tpidx_hbm, out_hbm,
                bufpp0, bufpp1, buftp0, buftp1, hist, sem0, sem1):
        wid = lax.axis_index("s") * nc + lax.axis_index("c")
        cnt = jnp.where(wid < extra, base_count + 1, base_count)

        zeros16 = jnp.zeros((16,), jnp.float32)

        def zb(t, carry):
            hist[pl.ds(t * 16, 16)] = zeros16
            return carry

        lax.fori_loop(0, 16 * _LS // 16, zb, 0, unroll=16)

        lane_offs = lax.iota(jnp.int32, 16) * _LS
        ones = jnp.ones((16,), jnp.float32)
        bufs = ((bufpp0, buftp0, sem0), (bufpp1, buftp1, sem1))

        def copies(j, bufpp, buftp, sem):
            k = wid + j * nw
            blk_i = k // rpb
            src0 = (blk_i * _C) * blk + (k % rpb) * w
            out = []
            for c in range(_C):
                out.append(pltpu.make_async_copy(
                    bins_hbm.at[pl.ds(src0 + c * blk, w)],
                    bufpp.at[pl.ds(c * w, w)], sem))
            out.append(pltpu.make_async_copy(
                tpidx_hbm.at[pl.ds(k * w, w)], buftp, sem))
            return out

        def fire(j, bufpp, buftp, sem):
            for cp in copies(j, bufpp, buftp, sem):
                cp.start()

        def drain(j, bufpp, buftp, sem):
            for cp in copies(j, bufpp, buftp, sem):
                cp.wait()

        def process(bufpp, buftp):
            for c in range(_C):
                offs_c = lane_offs + (c * (_T + 1))

                def body_pp(v, c2, offs_c=offs_c, c=c):
                    idx = bufpp[pl.ds(c * w + v * 16, 16)] + offs_c
                    plsc.addupdate_scatter(hist, [idx], ones)
                    return c2

                lax.fori_loop(0, w // 16, body_pp, 0, unroll=10)

            offs_tp = lane_offs + _HP

            def body_tp(v, c2):
                idx = buftp[pl.ds(v * 16, 16)] + offs_tp
                plsc.addupdate_scatter(hist, [idx], ones)
                return c2

            lax.fori_loop(0, w // 16, body_tp, 0, unroll=10)

        @pl.when(cnt > 0)
        def _prime():
            fire(0, *bufs[0])

        def pair_body(j2, carry):
            j0 = j2 * 2

            @pl.when(j0 + 1 < cnt)
            def _f1():
                fire(j0 + 1, *bufs[1])

            drain(j0, *bufs[0])
            process(bufs[0][0], bufs[0][1])

            @pl.when(j0 + 2 < cnt)
            def _f0():
                fire(j0 + 2, *bufs[0])

            @pl.when(j0 + 1 < cnt)
            def _p1():
                drain(j0 + 1, *bufs[1])
                process(bufs[1][0], bufs[1][1])

            return carry

        lax.fori_loop(0, (cnt + 1) // 2, pair_body, 0)

        pltpu.sync_copy(hist, out_hbm.at[pl.ds(wid * 16 * _LS, 16 * _LS)])

    return sc_hist(bins_flat, tpidx)


def _excl_rev_cumsum_ge(x, total):
    incl = x
    k = 1
    while k < _T + 1:
        pad = jnp.zeros((_C, k), jnp.float32)
        incl = incl + jnp.concatenate([pad, incl[:, :-k]], axis=1)
        k *= 2
    return total - incl + x


def _stage3_body(n, parts_ref, cm_ref, losssum_ref,
                 map_out, prec_out, rec_out, recall_out,
                 precision_out, acc_out, loss_out):
    cm = cm_ref[...].astype(jnp.float32)
    eye = (jax.lax.broadcasted_iota(jnp.int32, (_C, _C), 0)
           == jax.lax.broadcasted_iota(jnp.int32, (_C, _C), 1))
    cmdiag = jnp.where(eye, cm, 0.0)
    tpd_col = jnp.sum(cmdiag, axis=1, keepdims=True)
    tpd_row = jnp.sum(cmdiag, axis=0, keepdims=True)
    support = jnp.sum(cm, axis=1, keepdims=True)
    predcnt = jnp.sum(cm, axis=0, keepdims=True)
    recall_per = tpd_col / jnp.maximum(support, 1.0)
    prec_per = tpd_row / jnp.maximum(predcnt, 1.0)
    recall = jnp.sum(recall_per) / _C
    precision = jnp.sum(prec_per) / _C
    recall_out[...] = jnp.reshape(recall, (1, 1))
    precision_out[...] = jnp.reshape(precision, (1, 1))
    acc_out[...] = jnp.reshape(recall, (1, 1))

    x4 = jnp.reshape(parts_ref[...], (-1, 2, _C, _T + 1))
    totals = jnp.sum(x4[:, 0], axis=0)
    tp_totals = jnp.sum(x4[:, 1], axis=0)
    pos_total = jnp.sum(tp_totals, axis=1, keepdims=True)
    pp_curve = _excl_rev_cumsum_ge(totals, jnp.float32(n))[:, :_T]
    tp_curve = _excl_rev_cumsum_ge(tp_totals, pos_total)[:, :_T]

    prec_curve = tp_curve / jnp.maximum(pp_curve, 1.0)
    rec_curve = tp_curve / jnp.maximum(pos_total, 1.0)
    ones_col = jnp.ones((_C, 1), jnp.float32)
    zeros_col = jnp.zeros((_C, 1), jnp.float32)
    prec_out[...] = jnp.concatenate([prec_curve, ones_col], axis=1)
    rec_out[...] = jnp.concatenate([rec_curve, zeros_col], axis=1)
    rec_next = jnp.concatenate([rec_curve[:, 1:], zeros_col], axis=1)
    ap = jnp.sum((rec_curve - rec_next) * prec_curve, axis=1, keepdims=True)
    map_out[...] = jnp.reshape(jnp.sum(ap) / _C, (1, 1))
    loss_out[...] = losssum_ref[...] / jnp.float32(n)


def _stage3(parts3, cm, loss_sum, n):
    f32 = jnp.float32
    npart = parts3.shape[0]
    one = lambda: pl.BlockSpec((1, 1), lambda: (0, 0))
    return pl.pallas_call(
        functools.partial(_stage3_body, n),
        grid=(),
        in_specs=[
            pl.BlockSpec((npart, _C, _T + 1), lambda: (0, 0, 0)),
            pl.BlockSpec((_C, _C), lambda: (0, 0)),
            one(),
        ],
        out_specs=[
            one(), pl.BlockSpec((_C, _T + 1), lambda: (0, 0)),
            pl.BlockSpec((_C, _T + 1), lambda: (0, 0)),
            one(), one(), one(), one(),
        ],
        out_shape=(
            jax.ShapeDtypeStruct((1, 1), f32),
            jax.ShapeDtypeStruct((_C, _T + 1), f32),
            jax.ShapeDtypeStruct((_C, _T + 1), f32),
            jax.ShapeDtypeStruct((1, 1), f32),
            jax.ShapeDtypeStruct((1, 1), f32),
            jax.ShapeDtypeStruct((1, 1), f32),
            jax.ShapeDtypeStruct((1, 1), f32),
        ),
    )(parts3, cm, loss_sum)


def kernel(logits, labels):
    n, c = logits.shape
    assert c == _C
    b = _pick_block(n)
    nb = n // b
    x3 = logits.reshape(nb, b, _C).transpose(0, 2, 1)
    lab3 = labels.astype(jnp.int32).reshape(nb, 1, b)

    bins3, tpidx3, cm, loss_sum = _stage1(x3, lab3)
    parts = _sc_hist(bins3.reshape(-1), tpidx3.reshape(-1), n, b)
    parts3 = parts.reshape(-1, _C, _T + 1)
    (map_, prec_full, rec_full, recall, precision, accuracy,
     loss) = _stage3(parts3, cm, loss_sum, n)

    thresholds = jnp.linspace(0.0, 1.0, _T)
    return (cm, map_.reshape(()), prec_full, rec_full, thresholds,
            recall.reshape(()), precision.reshape(()), accuracy.reshape(()),
            loss.reshape(()))


# speedup vs baseline: 51.4627x; 1.0740x over previous
import functools

import jax
import jax.numpy as jnp
from jax import lax
from jax.experimental import pallas as pl
from jax.experimental.pallas import tpu as pltpu
from jax.experimental.pallas import tpu_sc as plsc

_C = 16
_T = 200
_HP = _C * (_T + 1)
_LS = 2 * _HP


def _pick_block(n: int) -> int:
    for b in (4000, 2500, 2000, 1600, 1000, 800, 500, 400, 200, 100, 50, 25, 8):
        if n % b == 0:
            return b
    return n


def _stage1_body(nb, lab_ref, x_ref,
                 bins_out, tpidx_out, cm_out, loss_out,
                 cm_acc, loss_acc):
    i = pl.program_id(0)

    @pl.when(i == 0)
    def _init():
        cm_acc[...] = jnp.zeros_like(cm_acc)
        loss_acc[0, 0] = 0.0

    x = x_ref[0]
    lab = lab_ref[0]
    b = x.shape[1]

    m = jnp.max(x, axis=0, keepdims=True)
    e = jnp.exp(x - m)
    z = jnp.sum(e, axis=0, keepdims=True)
    inv = jnp.float32(_T - 1) / z
    s = e * inv
    bins = jnp.clip(s.astype(jnp.int32), 0, _T - 1)
    bins_out[...] = bins[None]

    em = jnp.max(e, axis=0, keepdims=True)
    iota_c = jax.lax.broadcasted_iota(jnp.int32, (_C, b), 0)
    pick = jnp.min(jnp.where(e == em, iota_c, _C), axis=0, keepdims=True)
    ohp = (iota_c == pick).astype(jnp.float32)
    ohl = lab == iota_c
    dn = (((1,), (1,)), ((), ()))
    cm_acc[...] += jax.lax.dot_general(ohl.astype(jnp.float32), ohp, dn,
                                       preferred_element_type=jnp.float32)

    bins_true = jnp.sum(jnp.where(ohl, bins, 0), axis=0, keepdims=True)
    tpidx_out[...] = (lab * (_T + 1) + bins_true)[None]

    xt = jnp.sum(jnp.where(ohl, x, 0.0), axis=0, keepdims=True)
    loss_acc[0, 0] += jnp.sum((m + jnp.log(z)) - xt)

    @pl.when(i == nb - 1)
    def _fin():
        cm_out[...] = cm_acc[...].astype(jnp.int32)
        loss_out[...] = jnp.reshape(loss_acc[0, 0], (1, 1))


def _stage1(x3, lab3):
    nb, _, b = x3.shape
    f32 = jnp.float32
    grid_spec = pltpu.PrefetchScalarGridSpec(
        num_scalar_prefetch=0,
        grid=(nb,),
        in_specs=[
            pl.BlockSpec((1, 1, b), lambda i: (i, 0, 0)),
            pl.BlockSpec((1, _C, b), lambda i: (i, 0, 0)),
        ],
        out_specs=[
            pl.BlockSpec((1, _C, b), lambda i: (i, 0, 0)),
            pl.BlockSpec((1, 1, b), lambda i: (i, 0, 0)),
            pl.BlockSpec((_C, _C), lambda i: (0, 0)),
            pl.BlockSpec((1, 1), lambda i: (0, 0)),
        ],
        scratch_shapes=[
            pltpu.VMEM((_C, _C), f32),
            pltpu.SMEM((1, 1), f32),
        ],
    )
    return pl.pallas_call(
        functools.partial(_stage1_body, nb),
        grid_spec=grid_spec,
        out_shape=(
            jax.ShapeDtypeStruct((nb, _C, b), jnp.int32),
            jax.ShapeDtypeStruct((nb, 1, b), jnp.int32),
            jax.ShapeDtypeStruct((_C, _C), jnp.int32),
            jax.ShapeDtypeStruct((1, 1), f32),
        ),
    )(lab3, x3)


def _sc_hist(bins_flat, tpidx, n, blk):
    info = plsc.get_sparse_core_info()
    nc, ns = info.num_cores, info.num_subcores
    nw = nc * ns
    w = 800
    assert n % w == 0 and w % 16 == 0 and blk % w == 0
    rpb = blk // w
    nchunks_total = n // w
    base_count = nchunks_total // nw
    extra = nchunks_total - base_count * nw
    mesh = plsc.VectorSubcoreMesh(core_axis_name="c", subcore_axis_name="s")

    @functools.partial(
        pl.kernel, mesh=mesh,
        compiler_params=pltpu.CompilerParams(needs_layout_passes=False),
        out_type=jax.ShapeDtypeStruct((nw * _LS,), jnp.int32),
        scratch_types=[
            pltpu.VMEM((_C * w,), jnp.int32),
            pltpu.VMEM((_C * w,), jnp.int32),
            pltpu.VMEM((w,), jnp.int32),
            pltpu.VMEM((w,), jnp.int32),
            pltpu.VMEM((16 * _LS,), jnp.int32),
            pltpu.SemaphoreType.DMA,
            pltpu.SemaphoreType.DMA,
        ],
    )
    def sc_hist(bins_hbm, tpidx_hbm, out_hbm,
                bufpp0, bufpp1, buftp0, buftp1, hist, sem0, sem1):
        wid = lax.axis_index("s") * nc + lax.axis_index("c")
        cnt = jnp.where(wid < extra, base_count + 1, base_count)

        zeros16 = jnp.zeros((16,), jnp.int32)

        def zb(t, carry):
            hist[pl.ds(t * 16, 16)] = zeros16
            return carry

        lax.fori_loop(0, 16 * _LS // 16, zb, 0, unroll=16)

        lane_offs = lax.iota(jnp.int32, 16)
        ones = jnp.ones((16,), jnp.int32)
        bufs = ((bufpp0, buftp0, sem0), (bufpp1, buftp1, sem1))

        def copies(j, bufpp, buftp, sem):
            k = wid + j * nw
            blk_i = k // rpb
            src0 = (blk_i * _C) * blk + (k % rpb) * w
            out = []
            for c in range(_C):
                out.append(pltpu.make_async_copy(
                    bins_hbm.at[pl.ds(src0 + c * blk, w)],
                    bufpp.at[pl.ds(c * w, w)], sem))
            out.append(pltpu.make_async_copy(
                tpidx_hbm.at[pl.ds(k * w, w)], buftp, sem))
            return out

        def fire(j, bufpp, buftp, sem):
            for cp in copies(j, bufpp, buftp, sem):
                cp.start()

        def drain(j, bufpp, buftp, sem):
            for cp in copies(j, bufpp, buftp, sem):
                cp.wait()

        def process(bufpp, buftp):
            for c in range(_C):
                offs_c = lane_offs + (c * (_T + 1) * 16)

                def body_pp(v, c2, offs_c=offs_c, c=c):
                    idx = bufpp[pl.ds(c * w + v * 16, 16)] * 16 + offs_c
                    plsc.addupdate_scatter(hist, [idx], ones)
                    return c2

                lax.fori_loop(0, w // 16, body_pp, 0, unroll=10)

            offs_tp = lane_offs + _HP * 16

            def body_tp(v, c2):
                idx = buftp[pl.ds(v * 16, 16)] * 16 + offs_tp
                plsc.addupdate_scatter(hist, [idx], ones)
                return c2

            lax.fori_loop(0, w // 16, body_tp, 0, unroll=10)

        @pl.when(cnt > 0)
        def _prime():
            fire(0, *bufs[0])

        def pair_body(j2, carry):
            j0 = j2 * 2

            @pl.when(j0 + 1 < cnt)
            def _f1():
                fire(j0 + 1, *bufs[1])

            drain(j0, *bufs[0])
            process(bufs[0][0], bufs[0][1])

            @pl.when(j0 + 2 < cnt)
            def _f0():
                fire(j0 + 2, *bufs[0])

            @pl.when(j0 + 1 < cnt)
            def _p1():
                drain(j0 + 1, *bufs[1])
                process(bufs[1][0], bufs[1][1])

            return carry

        lax.fori_loop(0, (cnt + 1) // 2, pair_body, 0)

        idx16 = lane_offs * 16

        def fold_body(v, carry):
            acc = plsc.load_gather(hist, [idx16 + v * 256])
            for l in range(1, 16):
                acc = acc + plsc.load_gather(hist, [idx16 + (v * 256 + l)])
            bufpp0[pl.ds(v * 16, 16)] = acc
            return carry

        lax.fori_loop(0, _LS // 16, fold_body, 0, unroll=2)
        pltpu.sync_copy(bufpp0.at[pl.ds(0, _LS)], out_hbm.at[pl.ds(wid * _LS, _LS)])

    return sc_hist(bins_flat, tpidx)


def _excl_rev_cumsum_ge(x, total):
    incl = x
    k = 1
    while k < _T + 1:
        pad = jnp.zeros((_C, k), jnp.float32)
        incl = incl + jnp.concatenate([pad, incl[:, :-k]], axis=1)
        k *= 2
    return total - incl + x


def _stage3_body(n, parts_ref, cm_ref, losssum_ref,
                 map_out, prec_out, rec_out, recall_out,
                 precision_out, acc_out, loss_out):
    cm = cm_ref[...].astype(jnp.float32)
    eye = (jax.lax.broadcasted_iota(jnp.int32, (_C, _C), 0)
           == jax.lax.broadcasted_iota(jnp.int32, (_C, _C), 1))
    cmdiag = jnp.where(eye, cm, 0.0)
    tpd_col = jnp.sum(cmdiag, axis=1, keepdims=True)
    tpd_row = jnp.sum(cmdiag, axis=0, keepdims=True)
    support = jnp.sum(cm, axis=1, keepdims=True)
    predcnt = jnp.sum(cm, axis=0, keepdims=True)
    recall_per = tpd_col / jnp.maximum(support, 1.0)
    prec_per = tpd_row / jnp.maximum(predcnt, 1.0)
    recall = jnp.sum(recall_per) / _C
    precision = jnp.sum(prec_per) / _C
    recall_out[...] = jnp.reshape(recall, (1, 1))
    precision_out[...] = jnp.reshape(precision, (1, 1))
    acc_out[...] = jnp.reshape(recall, (1, 1))

    xl = parts_ref[...].astype(jnp.float32)
    x4 = jnp.reshape(xl, (-1, 2, _C, _T + 1))
    totals = jnp.sum(x4[:, 0], axis=0)
    tp_totals = jnp.sum(x4[:, 1], axis=0)
    pos_total = jnp.sum(tp_totals, axis=1, keepdims=True)
    pp_curve = _excl_rev_cumsum_ge(totals, jnp.float32(n))[:, :_T]
    tp_curve = _excl_rev_cumsum_ge(tp_totals, pos_total)[:, :_T]

    prec_curve = tp_curve / jnp.maximum(pp_curve, 1.0)
    rec_curve = tp_curve / jnp.maximum(pos_total, 1.0)
    ones_col = jnp.ones((_C, 1), jnp.float32)
    zeros_col = jnp.zeros((_C, 1), jnp.float32)
    prec_out[...] = jnp.concatenate([prec_curve, ones_col], axis=1)
    rec_out[...] = jnp.concatenate([rec_curve, zeros_col], axis=1)
    rec_next = jnp.concatenate([rec_curve[:, 1:], zeros_col], axis=1)
    ap = jnp.sum((rec_curve - rec_next) * prec_curve, axis=1, keepdims=True)
    map_out[...] = jnp.reshape(jnp.sum(ap) / _C, (1, 1))
    loss_out[...] = losssum_ref[...] / jnp.float32(n)


def _stage3(parts3, cm, loss_sum, n):
    f32 = jnp.float32
    npart = parts3.shape[0]
    one = lambda: pl.BlockSpec((1, 1), lambda: (0, 0))
    return pl.pallas_call(
        functools.partial(_stage3_body, n),
        grid=(),
        in_specs=[
            pl.BlockSpec((npart, _T + 1), lambda: (0, 0)),
            pl.BlockSpec((_C, _C), lambda: (0, 0)),
            one(),
        ],
        out_specs=[
            one(), pl.BlockSpec((_C, _T + 1), lambda: (0, 0)),
            pl.BlockSpec((_C, _T + 1), lambda: (0, 0)),
            one(), one(), one(), one(),
        ],
        out_shape=(
            jax.ShapeDtypeStruct((1, 1), f32),
            jax.ShapeDtypeStruct((_C, _T + 1), f32),
            jax.ShapeDtypeStruct((_C, _T + 1), f32),
            jax.ShapeDtypeStruct((1, 1), f32),
            jax.ShapeDtypeStruct((1, 1), f32),
            jax.ShapeDtypeStruct((1, 1), f32),
            jax.ShapeDtypeStruct((1, 1), f32),
        ),
    )(parts3, cm, loss_sum)


def kernel(logits, labels):
    n, c = logits.shape
    assert c == _C
    b = _pick_block(n)
    nb = n // b
    x3 = logits.reshape(nb, b, _C).transpose(0, 2, 1)
    lab3 = labels.astype(jnp.int32).reshape(nb, 1, b)

    bins3, tpidx3, cm, loss_sum = _stage1(x3, lab3)
    parts = _sc_hist(bins3.reshape(-1), tpidx3.reshape(-1), n, b)
    parts3 = parts.reshape(-1, _T + 1)
    (map_, prec_full, rec_full, recall, precision, accuracy,
     loss) = _stage3(parts3, cm, loss_sum, n)

    thresholds = jnp.linspace(0.0, 1.0, _T)
    return (cm, map_.reshape(()), prec_full, rec_full, thresholds,
            recall.reshape(()), precision.reshape(()), accuracy.reshape(()),
            loss.reshape(()))


# speedup vs baseline: 53.0235x; 1.0303x over previous
import functools

import jax
import jax.numpy as jnp
from jax import lax
from jax.experimental import pallas as pl
from jax.experimental.pallas import tpu as pltpu
from jax.experimental.pallas import tpu_sc as plsc

_C = 16
_T = 200
_HP = _C * (_T + 1)
_LS = 2 * _HP


def _pick_block(n: int) -> int:
    for b in (4000, 2500, 2000, 1600, 1000, 800, 500, 400, 200, 100, 50, 25, 8):
        if n % b == 0:
            return b
    return n


def _stage1_body(nb, lab_ref, x_ref,
                 bins_out, tpidx_out, cm_out, loss_out,
                 cm_acc, loss_acc):
    i = pl.program_id(0)

    @pl.when(i == 0)
    def _init():
        cm_acc[...] = jnp.zeros_like(cm_acc)
        loss_acc[0, 0] = 0.0

    x = x_ref[0]
    lab = lab_ref[0]
    b = x.shape[1]

    m = jnp.max(x, axis=0, keepdims=True)
    e = jnp.exp(x - m)
    z = jnp.sum(e, axis=0, keepdims=True)
    inv = jnp.float32(_T - 1) / z
    s = e * inv
    bins = jnp.clip(s.astype(jnp.int32), 0, _T - 1)
    iota_cb = jax.lax.broadcasted_iota(jnp.int32, (_C, b), 0)
    bins_out[...] = (bins * 16 + iota_cb * ((_T + 1) * 16))[None]

    em = jnp.max(e, axis=0, keepdims=True)
    iota_c = jax.lax.broadcasted_iota(jnp.int32, (_C, b), 0)
    pick = jnp.min(jnp.where(e == em, iota_c, _C), axis=0, keepdims=True)
    ohp = (iota_c == pick).astype(jnp.float32)
    ohl = lab == iota_c
    dn = (((1,), (1,)), ((), ()))
    cm_acc[...] += jax.lax.dot_general(ohl.astype(jnp.float32), ohp, dn,
                                       preferred_element_type=jnp.float32)

    bins_true = jnp.sum(jnp.where(ohl, bins, 0), axis=0, keepdims=True)
    tpidx_out[...] = ((lab * (_T + 1) + bins_true) * 16 + _HP * 16)[None]

    xt = jnp.sum(jnp.where(ohl, x, 0.0), axis=0, keepdims=True)
    loss_acc[0, 0] += jnp.sum((m + jnp.log(z)) - xt)

    @pl.when(i == nb - 1)
    def _fin():
        cm_out[...] = cm_acc[...].astype(jnp.int32)
        loss_out[...] = jnp.reshape(loss_acc[0, 0], (1, 1))


def _stage1(x3, lab3):
    nb, _, b = x3.shape
    f32 = jnp.float32
    grid_spec = pltpu.PrefetchScalarGridSpec(
        num_scalar_prefetch=0,
        grid=(nb,),
        in_specs=[
            pl.BlockSpec((1, 1, b), lambda i: (i, 0, 0)),
            pl.BlockSpec((1, _C, b), lambda i: (i, 0, 0)),
        ],
        out_specs=[
            pl.BlockSpec((1, _C, b), lambda i: (i, 0, 0)),
            pl.BlockSpec((1, 1, b), lambda i: (i, 0, 0)),
            pl.BlockSpec((_C, _C), lambda i: (0, 0)),
            pl.BlockSpec((1, 1), lambda i: (0, 0)),
        ],
        scratch_shapes=[
            pltpu.VMEM((_C, _C), f32),
            pltpu.SMEM((1, 1), f32),
        ],
    )
    return pl.pallas_call(
        functools.partial(_stage1_body, nb),
        grid_spec=grid_spec,
        out_shape=(
            jax.ShapeDtypeStruct((nb, _C, b), jnp.int32),
            jax.ShapeDtypeStruct((nb, 1, b), jnp.int32),
            jax.ShapeDtypeStruct((_C, _C), jnp.int32),
            jax.ShapeDtypeStruct((1, 1), f32),
        ),
    )(lab3, x3)


def _sc_hist(bins_flat, tpidx, n, blk):
    info = plsc.get_sparse_core_info()
    nc, ns = info.num_cores, info.num_subcores
    nw = nc * ns
    w = 800
    assert n % w == 0 and w % 16 == 0 and blk % w == 0
    rpb = blk // w
    nchunks_total = n // w
    base_count = nchunks_total // nw
    extra = nchunks_total - base_count * nw
    mesh = plsc.VectorSubcoreMesh(core_axis_name="c", subcore_axis_name="s")

    @functools.partial(
        pl.kernel, mesh=mesh,
        compiler_params=pltpu.CompilerParams(needs_layout_passes=False),
        out_type=jax.ShapeDtypeStruct((nw * _LS,), jnp.int32),
        scratch_types=[
            pltpu.VMEM((_C * w,), jnp.int32),
            pltpu.VMEM((_C * w,), jnp.int32),
            pltpu.VMEM((w,), jnp.int32),
            pltpu.VMEM((w,), jnp.int32),
            pltpu.VMEM((16 * _LS,), jnp.int32),
            pltpu.SemaphoreType.DMA,
            pltpu.SemaphoreType.DMA,
        ],
    )
    def sc_hist(bins_hbm, tpidx_hbm, out_hbm,
                bufpp0, bufpp1, buftp0, buftp1, hist, sem0, sem1):
        wid = lax.axis_index("s") * nc + lax.axis_index("c")
        cnt = jnp.where(wid < extra, base_count + 1, base_count)

        zeros16 = jnp.zeros((16,), jnp.int32)

        def zb(t, carry):
            hist[pl.ds(t * 16, 16)] = zeros16
            return carry

        lax.fori_loop(0, 16 * _LS // 16, zb, 0, unroll=16)

        lane_offs = lax.iota(jnp.int32, 16)
        ones = jnp.ones((16,), jnp.int32)
        bufs = ((bufpp0, buftp0, sem0), (bufpp1, buftp1, sem1))

        def copies(j, bufpp, buftp, sem):
            k = wid + j * nw
            blk_i = k // rpb
            src0 = (blk_i * _C) * blk + (k % rpb) * w
            out = []
            for c in range(_C):
                out.append(pltpu.make_async_copy(
                    bins_hbm.at[pl.ds(src0 + c * blk, w)],
                    bufpp.at[pl.ds(c * w, w)], sem))
            out.append(pltpu.make_async_copy(
                tpidx_hbm.at[pl.ds(k * w, w)], buftp, sem))
            return out

        def fire(j, bufpp, buftp, sem):
            for cp in copies(j, bufpp, buftp, sem):
                cp.start()

        def drain(j, bufpp, buftp, sem):
            for cp in copies(j, bufpp, buftp, sem):
                cp.wait()

        def process(bufpp, buftp):
            def body_pp(v, c2):
                idx = bufpp[pl.ds(v * 16, 16)] + lane_offs
                plsc.addupdate_scatter(hist, [idx], ones)
                return c2

            lax.fori_loop(0, _C * w // 16, body_pp, 0, unroll=16)

            def body_tp(v, c2):
                idx = buftp[pl.ds(v * 16, 16)] + lane_offs
                plsc.addupdate_scatter(hist, [idx], ones)
                return c2

            lax.fori_loop(0, w // 16, body_tp, 0, unroll=10)

        @pl.when(cnt > 0)
        def _prime():
            fire(0, *bufs[0])

        def pair_body(j2, carry):
            j0 = j2 * 2

            @pl.when(j0 + 1 < cnt)
            def _f1():
                fire(j0 + 1, *bufs[1])

            drain(j0, *bufs[0])
            process(bufs[0][0], bufs[0][1])

            @pl.when(j0 + 2 < cnt)
            def _f0():
                fire(j0 + 2, *bufs[0])

            @pl.when(j0 + 1 < cnt)
            def _p1():
                drain(j0 + 1, *bufs[1])
                process(bufs[1][0], bufs[1][1])

            return carry

        lax.fori_loop(0, (cnt + 1) // 2, pair_body, 0)

        idx16 = lane_offs * 16

        def fold_body(v, carry):
            acc = plsc.load_gather(hist, [idx16 + v * 256])
            for l in range(1, 16):
                acc = acc + plsc.load_gather(hist, [idx16 + (v * 256 + l)])
            bufpp0[pl.ds(v * 16, 16)] = acc
            return carry

        lax.fori_loop(0, _LS // 16, fold_body, 0, unroll=2)
        pltpu.sync_copy(bufpp0.at[pl.ds(0, _LS)], out_hbm.at[pl.ds(wid * _LS, _LS)])

    return sc_hist(bins_flat, tpidx)


def _excl_rev_cumsum_ge(x, total):
    incl = x
    k = 1
    while k < _T + 1:
        pad = jnp.zeros((_C, k), jnp.float32)
        incl = incl + jnp.concatenate([pad, incl[:, :-k]], axis=1)
        k *= 2
    return total - incl + x


def _stage3_body(n, parts_ref, cm_ref, losssum_ref,
                 map_out, prec_out, rec_out, recall_out,
                 precision_out, acc_out, loss_out):
    cm = cm_ref[...].astype(jnp.float32)
    eye = (jax.lax.broadcasted_iota(jnp.int32, (_C, _C), 0)
           == jax.lax.broadcasted_iota(jnp.int32, (_C, _C), 1))
    cmdiag = jnp.where(eye, cm, 0.0)
    tpd_col = jnp.sum(cmdiag, axis=1, keepdims=True)
    tpd_row = jnp.sum(cmdiag, axis=0, keepdims=True)
    support = jnp.sum(cm, axis=1, keepdims=True)
    predcnt = jnp.sum(cm, axis=0, keepdims=True)
    recall_per = tpd_col / jnp.maximum(support, 1.0)
    prec_per = tpd_row / jnp.maximum(predcnt, 1.0)
    recall = jnp.sum(recall_per) / _C
    precision = jnp.sum(prec_per) / _C
    recall_out[...] = jnp.reshape(recall, (1, 1))
    precision_out[...] = jnp.reshape(precision, (1, 1))
    acc_out[...] = jnp.reshape(recall, (1, 1))

    xl = parts_ref[...].astype(jnp.float32)
    x4 = jnp.reshape(xl, (-1, 2, _C, _T + 1))
    totals = jnp.sum(x4[:, 0], axis=0)
    tp_totals = jnp.sum(x4[:, 1], axis=0)
    pos_total = jnp.sum(tp_totals, axis=1, keepdims=True)
    pp_curve = _excl_rev_cumsum_ge(totals, jnp.float32(n))[:, :_T]
    tp_curve = _excl_rev_cumsum_ge(tp_totals, pos_total)[:, :_T]

    prec_curve = tp_curve / jnp.maximum(pp_curve, 1.0)
    rec_curve = tp_curve / jnp.maximum(pos_total, 1.0)
    ones_col = jnp.ones((_C, 1), jnp.float32)
    zeros_col = jnp.zeros((_C, 1), jnp.float32)
    prec_out[...] = jnp.concatenate([prec_curve, ones_col], axis=1)
    rec_out[...] = jnp.concatenate([rec_curve, zeros_col], axis=1)
    rec_next = jnp.concatenate([rec_curve[:, 1:], zeros_col], axis=1)
    ap = jnp.sum((rec_curve - rec_next) * prec_curve, axis=1, keepdims=True)
    map_out[...] = jnp.reshape(jnp.sum(ap) / _C, (1, 1))
    loss_out[...] = losssum_ref[...] / jnp.float32(n)


def _stage3(parts3, cm, loss_sum, n):
    f32 = jnp.float32
    npart = parts3.shape[0]
    one = lambda: pl.BlockSpec((1, 1), lambda: (0, 0))
    return pl.pallas_call(
        functools.partial(_stage3_body, n),
        grid=(),
        in_specs=[
            pl.BlockSpec((npart, _T + 1), lambda: (0, 0)),
            pl.BlockSpec((_C, _C), lambda: (0, 0)),
            one(),
        ],
        out_specs=[
            one(), pl.BlockSpec((_C, _T + 1), lambda: (0, 0)),
            pl.BlockSpec((_C, _T + 1), lambda: (0, 0)),
            one(), one(), one(), one(),
        ],
        out_shape=(
            jax.ShapeDtypeStruct((1, 1), f32),
            jax.ShapeDtypeStruct((_C, _T + 1), f32),
            jax.ShapeDtypeStruct((_C, _T + 1), f32),
            jax.ShapeDtypeStruct((1, 1), f32),
            jax.ShapeDtypeStruct((1, 1), f32),
            jax.ShapeDtypeStruct((1, 1), f32),
            jax.ShapeDtypeStruct((1, 1), f32),
        ),
    )(parts3, cm, loss_sum)


def kernel(logits, labels):
    n, c = logits.shape
    assert c == _C
    b = _pick_block(n)
    nb = n // b
    x3 = logits.reshape(nb, b, _C).transpose(0, 2, 1)
    lab3 = labels.astype(jnp.int32).reshape(nb, 1, b)

    bins3, tpidx3, cm, loss_sum = _stage1(x3, lab3)
    parts = _sc_hist(bins3.reshape(-1), tpidx3.reshape(-1), n, b)
    parts3 = parts.reshape(-1, _T + 1)
    (map_, prec_full, rec_full, recall, precision, accuracy,
     loss) = _stage3(parts3, cm, loss_sum, n)

    thresholds = jnp.linspace(0.0, 1.0, _T)
    return (cm, map_.reshape(()), prec_full, rec_full, thresholds,
            recall.reshape(()), precision.reshape(()), accuracy.reshape(()),
            loss.reshape(()))
